# baseline (speedup 1.0000x reference)
import functools

import jax
import jax.numpy as jnp
import numpy as np
from jax import lax
from jax.experimental import pallas as pl
from jax.experimental.pallas import tpu as pltpu
from jax.experimental.pallas import tpu_sc as plsc

NC = 2
NS = 16
NBUF = 4


LP = 128


def _make_pool_kernel(B, L, V, E):
    NW = NC * NS
    b_per_w = B // NW
    b_per_sc = B // NC
    assert b_per_w % NBUF == 0
    LS = -(-L // 8) * 8

    mesh = plsc.VectorSubcoreMesh(core_axis_name="c", subcore_axis_name="s",
                                  num_cores=NC, num_subcores=NS)

    @functools.partial(
        pl.kernel,
        out_type=jax.ShapeDtypeStruct((B, E), jnp.float32),
        mesh=mesh,
        compiler_params=pltpu.CompilerParams(use_tc_tiling_on_sc=False),
        scratch_types=[
            pltpu.VMEM((b_per_w, LP), jnp.int32),
            pltpu.VMEM((b_per_w, LP), jnp.int32),
            pltpu.VMEM((NBUF, LS, E), jnp.float32),
            pltpu.VMEM((b_per_w, E), jnp.float32),
            pltpu.VMEM_SHARED((b_per_sc, E), jnp.float32),
            [pltpu.SemaphoreType.DMA] * NBUF,
        ],
    )
    def pool(x_hbm, dest_hbm, table_hbm, out_hbm,
             idx_all, dst_all, rows_v, zbuf, pooled_s, sems):
        c = lax.axis_index("c")
        s = lax.axis_index("s")
        w = c * NS + s
        row0 = w * b_per_w

        pltpu.sync_copy(x_hbm.at[pl.ds(row0, b_per_w)], idx_all)
        pltpu.sync_copy(dest_hbm.at[pl.ds(row0, b_per_w)], dst_all)

        zeros16 = jnp.zeros((16,), jnp.float32)

        def zrow(i, carry):
            for j in range(E // 16):
                zbuf[i, pl.ds(j * 16, 16)] = zeros16
            return carry

        lax.fori_loop(0, b_per_w, zrow, 0)
        pltpu.sync_copy(zbuf, pooled_s.at[pl.ds(s * b_per_w, b_per_w)])

        def start_gather(b, slot):
            pltpu.async_copy(table_hbm.at[idx_all.at[b, pl.ds(0, LS)]],
                             rows_v.at[slot], sems[slot])

        def wait_gather(b, slot):
            pltpu.make_async_copy(table_hbm.at[idx_all.at[b, pl.ds(0, LS)]],
                                  rows_v.at[slot], sems[slot]).wait()

        def scatter_add(b, slot):
            pltpu.sync_copy(rows_v.at[slot],
                            pooled_s.at[dst_all.at[b, pl.ds(0, LS)]],
                            add=True)

        for k in range(NBUF):
            start_gather(k, k)

        def step(i, carry):
            g = i * NBUF
            for k in range(NBUF):
                b = g + k
                wait_gather(b, k)
                scatter_add(b, k)

                @pl.when(b + NBUF < b_per_w)
                def _():
                    start_gather(b + NBUF, k)
            return carry

        lax.fori_loop(0, b_per_w // NBUF, step, 0)

        pltpu.sync_copy(pooled_s.at[pl.ds(s * b_per_w, b_per_w)],
                        out_hbm.at[pl.ds(row0, b_per_w)])

    return pool


def _ffn(pooled_sum, W1, b1, W2, b2, L):
    B, E = pooled_sum.shape
    HID = W1.shape[0]
    NCLS = W2.shape[0]
    blk = 512
    inv_l = np.float32(1.0 / L)

    def body(p_ref, w1_ref, b1_ref, w2_ref, b2_ref, o_ref):
        p = p_ref[...] * inv_l
        h = lax.dot_general(p, w1_ref[...], (((1,), (1,)), ((), ())),
                            preferred_element_type=jnp.float32)
        h = jnp.maximum(h + b1_ref[...], 0.0)
        o = lax.dot_general(h, w2_ref[...], (((1,), (1,)), ((), ())),
                            preferred_element_type=jnp.float32)
        o_ref[...] = o + b2_ref[...]

    return pl.pallas_call(
        body,
        grid=(B // blk,),
        in_specs=[
            pl.BlockSpec((blk, E), lambda i: (i, 0)),
            pl.BlockSpec((HID, E), lambda i: (0, 0)),
            pl.BlockSpec((1, HID), lambda i: (0, 0)),
            pl.BlockSpec((NCLS, HID), lambda i: (0, 0)),
            pl.BlockSpec((1, NCLS), lambda i: (0, 0)),
        ],
        out_specs=pl.BlockSpec((blk, NCLS), lambda i: (i, 0)),
        out_shape=jax.ShapeDtypeStruct((B, NCLS), jnp.float32),
    )(pooled_sum, W1, b1.reshape(1, HID), W2, b2.reshape(1, NCLS))


def kernel(x, table, W1, b1, W2, b2):
    B, L = x.shape
    V, E = table.shape
    x128 = jnp.pad(x.astype(jnp.int32), ((0, 0), (0, LP - L)))
    dest = jnp.asarray(
        np.broadcast_to((np.arange(B, dtype=np.int32) % (B // NC))[:, None],
                        (B, LP)))
    pool = _make_pool_kernel(B, L, V, E)
    pooled_sum = pool(x128, dest, table)
    return _ffn(pooled_sum, W1, b1, W2, b2, L)


# speedup vs baseline: 5.2879x; 5.2879x over previous
import functools

import jax
import jax.numpy as jnp
import numpy as np
from jax import lax
from jax.experimental import pallas as pl
from jax.experimental.pallas import tpu as pltpu
from jax.experimental.pallas import tpu_sc as plsc

NC = 2
NS = 16
NBUF = 4


def _make_pool_kernel(B, L, V2, E):
    NW = NC * NS
    b_per_w = B // NW
    assert b_per_w % NBUF == 0
    NL = E // 16
    UNROLL = 5
    assert L % UNROLL == 0

    mesh = plsc.VectorSubcoreMesh(core_axis_name="c", subcore_axis_name="s",
                                  num_cores=NC, num_subcores=NS)

    @functools.partial(
        pl.kernel,
        out_type=jax.ShapeDtypeStruct((B, E), jnp.float32),
        mesh=mesh,
        compiler_params=pltpu.CompilerParams(use_tc_tiling_on_sc=False),
        scratch_types=[
            pltpu.VMEM((b_per_w, L), jnp.int32),
            pltpu.VMEM((NBUF, L, E), jnp.float32),
            pltpu.VMEM((b_per_w, E), jnp.float32),
            [pltpu.SemaphoreType.DMA] * NBUF,
        ],
    )
    def pool(x_hbm, table_hbm, out_hbm, idx_all, rows_v, pooled_v, sems):
        c = lax.axis_index("c")
        s = lax.axis_index("s")
        w = c * NS + s
        row0 = w * b_per_w

        pltpu.sync_copy(x_hbm.at[pl.ds(row0, b_per_w)], idx_all)

        def start_gather(b, slot):
            pltpu.async_copy(table_hbm.at[idx_all.at[b]], rows_v.at[slot],
                             sems[slot])

        def wait_gather(b, slot):
            pltpu.make_async_copy(table_hbm.at[idx_all.at[b]],
                                  rows_v.at[slot], sems[slot]).wait()

        def accumulate(b, slot):
            zeros16 = jnp.zeros((16,), jnp.float32)

            def tok(t, accs):
                t0 = t * UNROLL
                out = list(accs)
                for u in range(UNROLL):
                    for j in range(NL):
                        out[j] = out[j] + rows_v[slot, t0 + u,
                                                 pl.ds(j * 16, 16)]
                return tuple(out)

            accs = lax.fori_loop(0, L // UNROLL, tok, (zeros16,) * NL)
            for j in range(NL):
                pooled_v[b, pl.ds(j * 16, 16)] = accs[j]

        for k in range(NBUF):
            start_gather(k, k)

        def step(i, carry):
            g = i * NBUF
            for k in range(NBUF):
                b = g + k
                wait_gather(b, k)

                @pl.when(b + NBUF < b_per_w)
                def _():
                    start_gather(b + NBUF, k)

                accumulate(b, k)
            return carry

        lax.fori_loop(0, b_per_w // NBUF, step, 0)

        pltpu.sync_copy(pooled_v, out_hbm.at[pl.ds(row0, b_per_w)])

    return pool


def _ffn(pooled_sum, W1, b1, W2, b2, L):
    B, E = pooled_sum.shape
    HID = W1.shape[0]
    NCLS = W2.shape[0]
    blk = 512
    inv_l = np.float32(1.0 / L)

    def body(p_ref, w1_ref, b1_ref, w2_ref, b2_ref, o_ref):
        p = p_ref[...] * inv_l
        h = lax.dot_general(p, w1_ref[...], (((1,), (1,)), ((), ())),
                            preferred_element_type=jnp.float32)
        h = jnp.maximum(h + b1_ref[...], 0.0)
        o = lax.dot_general(h, w2_ref[...], (((1,), (1,)), ((), ())),
                            preferred_element_type=jnp.float32)
        o_ref[...] = o + b2_ref[...]

    return pl.pallas_call(
        body,
        grid=(B // blk,),
        in_specs=[
            pl.BlockSpec((blk, E), lambda i: (i, 0)),
            pl.BlockSpec((HID, E), lambda i: (0, 0)),
            pl.BlockSpec((1, HID), lambda i: (0, 0)),
            pl.BlockSpec((NCLS, HID), lambda i: (0, 0)),
            pl.BlockSpec((1, NCLS), lambda i: (0, 0)),
        ],
        out_specs=pl.BlockSpec((blk, NCLS), lambda i: (i, 0)),
        out_shape=jax.ShapeDtypeStruct((B, NCLS), jnp.float32),
    )(pooled_sum, W1, b1.reshape(1, HID), W2, b2.reshape(1, NCLS))


def kernel(x, table, W1, b1, W2, b2):
    B, L = x.shape
    V, E = table.shape
    assert 128 % E == 0
    stride = 128 // E
    table2 = jnp.pad(table, ((0, 0), (0, 128 - E))).reshape(stride * V, E)
    x2 = x.astype(jnp.int32) * stride
    pool = _make_pool_kernel(B, L, stride * V, E)
    pooled_sum = pool(x2, table2)
    return _ffn(pooled_sum, W1, b1, W2, b2, L)


# speedup vs baseline: 5.3912x; 1.0195x over previous
import functools

import jax
import jax.numpy as jnp
import numpy as np
from jax import lax
from jax.experimental import pallas as pl
from jax.experimental.pallas import tpu as pltpu
from jax.experimental.pallas import tpu_sc as plsc

NC = 2
NS = 16
NBUF = 4


def _make_pool_kernel(B, L, V2, E):
    NW = NC * NS
    b_per_w = B // NW
    assert b_per_w % NBUF == 0
    NL = E // 16
    UNROLL = 10
    assert L % UNROLL == 0

    mesh = plsc.VectorSubcoreMesh(core_axis_name="c", subcore_axis_name="s",
                                  num_cores=NC, num_subcores=NS)

    @functools.partial(
        pl.kernel,
        out_type=jax.ShapeDtypeStruct((B, E), jnp.float32),
        mesh=mesh,
        compiler_params=pltpu.CompilerParams(use_tc_tiling_on_sc=False),
        scratch_types=[
            pltpu.VMEM((b_per_w, L), jnp.int32),
            pltpu.VMEM((NBUF, L, E), jnp.float32),
            pltpu.VMEM((b_per_w, E), jnp.float32),
            [pltpu.SemaphoreType.DMA] * NBUF,
        ],
    )
    def pool(x_hbm, table_hbm, out_hbm, idx_all, rows_v, pooled_v, sems):
        c = lax.axis_index("c")
        s = lax.axis_index("s")
        w = c * NS + s
        row0 = w * b_per_w

        pltpu.sync_copy(x_hbm.at[pl.ds(row0, b_per_w)], idx_all)

        def start_gather(b, slot):
            pltpu.async_copy(table_hbm.at[idx_all.at[b]], rows_v.at[slot],
                             sems[slot])

        def wait_gather(b, slot):
            pltpu.make_async_copy(table_hbm.at[idx_all.at[b]],
                                  rows_v.at[slot], sems[slot]).wait()

        def accumulate(b, slot):
            zeros16 = jnp.zeros((16,), jnp.float32)

            def tok(t, accs):
                t0 = t * UNROLL
                out = list(accs)
                for u in range(UNROLL):
                    bank = (u % 2) * NL
                    for j in range(NL):
                        out[bank + j] = out[bank + j] + rows_v[
                            slot, t0 + u, pl.ds(j * 16, 16)]
                return tuple(out)

            accs = lax.fori_loop(0, L // UNROLL, tok, (zeros16,) * (2 * NL))
            for j in range(NL):
                pooled_v[b, pl.ds(j * 16, 16)] = accs[j] + accs[NL + j]

        for k in range(NBUF):
            start_gather(k, k)

        def step(i, carry):
            g = i * NBUF
            for k in range(NBUF):
                b = g + k
                wait_gather(b, k)

                @pl.when(b + NBUF < b_per_w)
                def _():
                    start_gather(b + NBUF, k)

                accumulate(b, k)
            return carry

        lax.fori_loop(0, b_per_w // NBUF, step, 0)

        pltpu.sync_copy(pooled_v, out_hbm.at[pl.ds(row0, b_per_w)])

    return pool


def _ffn(pooled_sum, W1, b1, W2, b2, L):
    B, E = pooled_sum.shape
    HID = W1.shape[0]
    NCLS = W2.shape[0]
    blk = 1024
    inv_l = np.float32(1.0 / L)

    def body(p_ref, w1_ref, b1_ref, w2_ref, b2_ref, o_ref):
        p = p_ref[...] * inv_l
        h = lax.dot_general(p, w1_ref[...], (((1,), (1,)), ((), ())),
                            preferred_element_type=jnp.float32)
        h = jnp.maximum(h + b1_ref[...], 0.0)
        o = lax.dot_general(h, w2_ref[...], (((1,), (1,)), ((), ())),
                            preferred_element_type=jnp.float32)
        o_ref[...] = o + b2_ref[...]

    return pl.pallas_call(
        body,
        grid=(B // blk,),
        in_specs=[
            pl.BlockSpec((blk, E), lambda i: (i, 0)),
            pl.BlockSpec((HID, E), lambda i: (0, 0)),
            pl.BlockSpec((1, HID), lambda i: (0, 0)),
            pl.BlockSpec((NCLS, HID), lambda i: (0, 0)),
            pl.BlockSpec((1, NCLS), lambda i: (0, 0)),
        ],
        out_specs=pl.BlockSpec((blk, NCLS), lambda i: (i, 0)),
        out_shape=jax.ShapeDtypeStruct((B, NCLS), jnp.float32),
    )(pooled_sum, W1, b1.reshape(1, HID), W2, b2.reshape(1, NCLS))


def kernel(x, table, W1, b1, W2, b2):
    B, L = x.shape
    V, E = table.shape
    assert 128 % E == 0
    stride = 128 // E
    table2 = jnp.pad(table, ((0, 0), (0, 128 - E))).reshape(stride * V, E)
    x2 = x.astype(jnp.int32) * stride
    pool = _make_pool_kernel(B, L, stride * V, E)
    pooled_sum = pool(x2, table2)
    return _ffn(pooled_sum, W1, b1, W2, b2, L)
